# baseline (speedup 1.0000x reference)
import jax
import jax.numpy as jnp
from jax.experimental import pallas as pl
from jax.experimental.pallas import tpu as pltpu

_ROWS, _COLS = 500000, 128
_NCHUNKS = 8
_CHUNK = _ROWS // _NCHUNKS


def _copy_body(in_ref, out_ref, *sems):
    for i in range(_NCHUNKS):
        pltpu.make_async_copy(
            in_ref.at[pl.ds(i * _CHUNK, _CHUNK), :],
            out_ref.at[pl.ds(i * _CHUNK, _CHUNK), :],
            sems[i],
        ).start()
    for i in range(_NCHUNKS):
        pltpu.make_async_copy(
            in_ref.at[pl.ds(i * _CHUNK, _CHUNK), :],
            out_ref.at[pl.ds(i * _CHUNK, _CHUNK), :],
            sems[i],
        ).wait()


def kernel(embeds):
    x = embeds.reshape(_ROWS, _COLS)
    out = pl.pallas_call(
        _copy_body,
        in_specs=[pl.BlockSpec(memory_space=pl.ANY)],
        out_specs=pl.BlockSpec(memory_space=pl.ANY),
        out_shape=jax.ShapeDtypeStruct((_ROWS, _COLS), jnp.float32),
        scratch_shapes=[pltpu.SemaphoreType.DMA] * _NCHUNKS,
    )(x)
    return out.reshape(embeds.shape)


# speedup vs baseline: 6.5528x; 6.5528x over previous
import jax
import jax.numpy as jnp
from jax.experimental import pallas as pl


def _copy_body(in_ref, out_ref):
    out_ref[...] = in_ref[...]


def kernel(embeds):
    rows, cols = 500000, 128
    block_rows = 25000
    x = embeds.reshape(rows, cols)
    out = pl.pallas_call(
        _copy_body,
        grid=(rows // block_rows,),
        in_specs=[pl.BlockSpec((block_rows, cols), lambda i: (i, 0))],
        out_specs=pl.BlockSpec((block_rows, cols), lambda i: (i, 0)),
        out_shape=jax.ShapeDtypeStruct((rows, cols), jnp.float32),
    )(x)
    return out.reshape(embeds.shape)


# speedup vs baseline: 8.9417x; 1.3646x over previous
import jax
import jax.numpy as jnp
from jax.experimental import pallas as pl


def _copy_body(in_ref, out_ref):
    out_ref[...] = in_ref[...]


def kernel(embeds):
    rows, cols = embeds.shape
    block_rows = 10000
    return pl.pallas_call(
        _copy_body,
        grid=(rows // block_rows,),
        in_specs=[pl.BlockSpec((block_rows, cols), lambda i: (i, 0))],
        out_specs=pl.BlockSpec((block_rows, cols), lambda i: (i, 0)),
        out_shape=jax.ShapeDtypeStruct((rows, cols), embeds.dtype),
    )(embeds)
